# baseline (speedup 1.0000x reference)
import functools

import jax
import jax.numpy as jnp
from jax import lax
from jax.experimental import pallas as pl
from jax.experimental.pallas import tpu as pltpu
from jax.experimental.pallas import tpu_sc as plsc

N = 50000
E = 800000
D = 64
M = 1024
NC = 2
NS = 16
LANES = 16

HALF = N // NC
STRIPE = 1560
SP_ROWS = 25008
LAST_STRIPE = HALF - (NS - 1) * STRIPE
TRASH = HALF
CH = 128
NCHUNK_E = E // CH
NCHUNK_N = (N + CH - 1) // CH

MP = 1152
MTRASH = M
ZSTRIPE = MP // NS
WSTRIPE = M // NS
NCHUNK_P = (HALF + CH - 1) // CH

_mesh = plsc.VectorSubcoreMesh(core_axis_name="c", subcore_axis_name="s")
_sc_params = pltpu.CompilerParams(use_tc_tiling_on_sc=False)
_sc_params_nolayout = pltpu.CompilerParams(use_tc_tiling_on_sc=False,
                                           needs_layout_passes=False)


@functools.partial(
    pl.kernel,
    out_type=jax.ShapeDtypeStruct((N, D), jnp.float32),
    mesh=_mesh,
    compiler_params=_sc_params,
    scratch_types=[
        pltpu.VMEM((CH,), jnp.int32),
        pltpu.VMEM((CH, D), jnp.float32),
    ],
)
def _emb_kernel(emb_hbm, ids_hbm, x_hbm, idv, rows):
    c = lax.axis_index("c")
    s = lax.axis_index("s")
    w = s * NC + c

    @pl.loop(w, NCHUNK_N, step=NC * NS)
    def _(ci):
        b = pl.multiple_of(jnp.minimum(ci * CH, N - CH), 8)
        pltpu.sync_copy(ids_hbm.at[pl.ds(b, CH)], idv)
        pltpu.sync_copy(emb_hbm.at[idv], rows)
        pltpu.sync_copy(rows, x_hbm.at[pl.ds(b, CH)])


SUB_E = E // NS
ECH = 96
SUP_E = 8 * ECH
PSUPS = SUB_E // SUP_E + 1
SUP_CLAMP = SUB_E - SUP_E
QPS = SUP_E // ECH
NBUF = 4
SEGSUP = 68
SEG = SEGSUP * SUP_E
NSEG = NC * NS
STAGE_CAP = SUP_E + LANES


@functools.partial(
    pl.kernel,
    out_type=(jax.ShapeDtypeStruct((NSEG * SEG,), jnp.int32),
              jax.ShapeDtypeStruct((NSEG * SEG,), jnp.int32),
              jax.ShapeDtypeStruct((NSEG, LANES), jnp.int32)),
    mesh=_mesh,
    compiler_params=_sc_params_nolayout,
    scratch_types=[
        pltpu.VMEM((SUP_E,), jnp.int32),
        pltpu.VMEM((SUP_E,), jnp.int32),
        pltpu.VMEM((SUP_E,), jnp.int32),
        pltpu.VMEM((SUP_E,), jnp.int32),
        pltpu.VMEM((STAGE_CAP,), jnp.int32),
        pltpu.VMEM((STAGE_CAP,), jnp.int32),
        pltpu.VMEM((LANES,), jnp.int32),
        pltpu.SMEM((2,), jnp.int32),
        pltpu.SemaphoreType.DMA,
        pltpu.SemaphoreType.DMA,
    ],
)
def _part_kernel(src_hbm, dst_hbm, srcp_hbm, dlp_hbm, cnt_hbm,
                 sb0, db0, sb1, db1, stg_s, stg_d, cntv, smem, pi0, pi1):
    c = lax.axis_index("c")
    s = lax.axis_index("s")
    base_node = c * HALF
    ebase = s * SUB_E
    widx = c * NS + s
    segbase = widx * SEG
    lane = lax.iota(jnp.int32, LANES)
    inb = ((sb0, db0, pi0), (sb1, db1, pi1))
    smem[0] = 0
    smem[1] = 0

    def pstart(j, p):
        off = pl.multiple_of(
            ebase + jnp.minimum(j * SUP_E, SUP_CLAMP), 8)
        pltpu.async_copy(src_hbm.at[pl.ds(off, SUP_E)], inb[p][0], inb[p][2])
        pltpu.async_copy(dst_hbm.at[pl.ds(off, SUP_E)], inb[p][1], inb[p][2])

    def pwait(p):
        pltpu.make_async_copy(src_hbm.at[pl.ds(0, SUP_E)], inb[p][0],
                              inb[p][2]).wait()
        pltpu.make_async_copy(dst_hbm.at[pl.ds(0, SUP_E)], inb[p][1],
                              inb[p][2]).wait()

    def flush():
        w = smem[1]
        woff = pl.multiple_of(segbase + w * SUP_E, 8)
        pltpu.sync_copy(stg_s.at[pl.ds(0, SUP_E)],
                        srcp_hbm.at[pl.ds(woff, SUP_E)])
        pltpu.sync_copy(stg_d.at[pl.ds(0, SUP_E)],
                        dlp_hbm.at[pl.ds(woff, SUP_E)])
        smem[1] = w + 1

    def proc(j, p):
        off = pl.multiple_of(
            ebase + jnp.minimum(j * SUP_E, SUP_CLAMP), 8)
        start = ebase + j * SUP_E
        sb, db, _ = inb[p]
        for g in range(SUP_E // LANES):
            dv = db[pl.ds(g * LANES, LANES)]
            sv = sb[pl.ds(g * LANES, LANES)]
            dl = dv - base_node
            t = start - off - g * LANES
            ok = (dl >= 0) & (dl < HALF) & (lane >= t)
            cur = smem[0]
            plsc.store_compressed(stg_s.at[pl.ds(cur, LANES)], sv, mask=ok)
            plsc.store_compressed(stg_d.at[pl.ds(cur, LANES)], dl, mask=ok)
            cur = cur + jnp.max(plsc.all_reduce_population_count(ok))
            smem[0] = cur

            @pl.when(cur >= SUP_E)
            def _():
                flush()
                ts = stg_s[pl.ds(SUP_E, LANES)]
                td = stg_d[pl.ds(SUP_E, LANES)]
                stg_s[pl.ds(0, LANES)] = ts
                stg_d[pl.ds(0, LANES)] = td
                smem[0] = cur - SUP_E

    pstart(0, 0)

    @pl.loop(0, PSUPS, step=2)
    def _(j):
        pwait(0)
        pstart(j + 1, 1)
        proc(j, 0)
        pwait(1)

        @pl.when(j + 2 < PSUPS)
        def _():
            pstart(j + 2, 0)
        proc(j + 1, 1)

    cur = smem[0]

    @pl.loop(0, STAGE_CAP // LANES)
    def _(g):
        pos = g * LANES
        vs = stg_s[pl.ds(pos, LANES)]
        vd = stg_d[pl.ds(pos, LANES)]
        keep = (pos + lane) < cur
        stg_s[pl.ds(pos, LANES)] = jnp.where(keep, vs, 0)
        stg_d[pl.ds(pos, LANES)] = jnp.where(keep, vd, TRASH)
    flush()

    @pl.loop(0, SUP_E // LANES)
    def _(g):
        stg_s[pl.ds(g * LANES, LANES)] = jnp.zeros((LANES,), jnp.int32)
        stg_d[pl.ds(g * LANES, LANES)] = jnp.full((LANES,), TRASH, jnp.int32)

    @pl.when(smem[1] % 2 == 1)
    def _():
        flush()

    cntv[pl.ds(0, LANES)] = jnp.zeros((LANES,), jnp.int32) + smem[1]
    pltpu.sync_copy(cntv, cnt_hbm.at[widx])


@functools.partial(
    pl.kernel,
    out_type=jax.ShapeDtypeStruct((N, D), jnp.float32),
    mesh=_mesh,
    compiler_params=_sc_params_nolayout,
    scratch_types=[
        pltpu.VMEM_SHARED((SP_ROWS, D), jnp.float32),
        pltpu.VMEM((SUP_E,), jnp.int32),
        pltpu.VMEM((SUP_E,), jnp.int32),
        pltpu.VMEM((SUP_E,), jnp.int32),
        pltpu.VMEM((SUP_E,), jnp.int32),
    ] + [pltpu.VMEM((ECH,), jnp.int32) for _ in range(2 * QPS)]
    + [
        pltpu.VMEM((ECH, D), jnp.float32),
        pltpu.VMEM((ECH, D), jnp.float32),
        pltpu.VMEM((ECH, D), jnp.float32),
        pltpu.VMEM((ECH, D), jnp.float32),
        pltpu.VMEM((LANES,), jnp.int32),
        pltpu.SemaphoreType.DMA,
        pltpu.SemaphoreType.DMA,
        pltpu.SemaphoreType.DMA,
        pltpu.SemaphoreType.DMA,
        pltpu.SemaphoreType.DMA,
        pltpu.SemaphoreType.DMA,
        pltpu.SemaphoreType.DMA,
        pltpu.SemaphoreType.DMA,
        pltpu.SemaphoreType.DMA,
        pltpu.SemaphoreType.DMA,
    ],
)
def _msg_kernel(h_hbm, srcp_hbm, dlp_hbm, cnt_hbm, z_hbm, acc,
                ibs0, ibs1, ibd0, ibd1,
                d00, d01, d02, d03, d04, d05, d06, d07,
                d10, d11, d12, d13, d14, d15, d16, d17,
                r0, r1, r2, r3, cntv,
                si0, si1, sg0, sg1, sg2, sg3, sc0, sc1, sc2, sc3):
    c = lax.axis_index("c")
    s = lax.axis_index("s")
    base_node = c * HALF
    widx = c * NS + s
    segbase = widx * SEG
    rows = (r0, r1, r2, r3)
    sg = (sg0, sg1, sg2, sg3)
    scm = (sc0, sc1, sc2, sc3)
    ibs = (ibs0, ibs1)
    ibd = (ibd0, ibd1)
    si = (si0, si1)
    dlb = ((d00, d01, d02, d03, d04, d05, d06, d07),
           (d10, d11, d12, d13, d14, d15, d16, d17))

    pltpu.sync_copy(cnt_hbm.at[widx], cntv)
    nsup = jnp.max(cntv[...])

    @pl.when(s < NS - 1)
    def _():
        off = pl.multiple_of(s * STRIPE, 8)
        pltpu.sync_copy(h_hbm.at[pl.ds(base_node + off, STRIPE)],
                        acc.at[pl.ds(off, STRIPE)])

    @pl.when(s == NS - 1)
    def _():
        off = (NS - 1) * STRIPE
        pltpu.sync_copy(h_hbm.at[pl.ds(base_node + off, LAST_STRIPE)],
                        acc.at[pl.ds(off, LAST_STRIPE)])

    plsc.subcore_barrier()

    def start_idx(jsup, p):
        off = pl.multiple_of(segbase + jsup * SUP_E, 8)
        pltpu.async_copy(srcp_hbm.at[pl.ds(off, SUP_E)], ibs[p], si[p])
        pltpu.async_copy(dlp_hbm.at[pl.ds(off, SUP_E)], ibd[p], si[p])

    def wait_idx(p):
        pltpu.make_async_copy(srcp_hbm.at[pl.ds(0, SUP_E)], ibs[p],
                              si[p]).wait()
        pltpu.make_async_copy(dlp_hbm.at[pl.ds(0, SUP_E)], ibd[p],
                              si[p]).wait()

    def wait_scatter(b):
        pltpu.make_async_copy(rows[b], acc.at[dlb[0][0]], scm[b]).wait()

    def finish_and_scatter(qs, dl_ref):
        b = qs % NBUF
        pltpu.make_async_copy(h_hbm.at[ibs0.at[pl.ds(0, ECH)]],
                              rows[b], sg[b]).wait()
        pltpu.async_copy(rows[b], acc.at[dl_ref], scm[b], add=True)

    STAG = 2

    def do_super(jsup, p, maybe_first):
        srcb = ibs[p]
        for q in range(QPS):
            b = q % NBUF
            qq = (q - STAG) % QPS
            dl_ref = dlb[1 - p][qq] if q < STAG else dlb[p][qq]
            if maybe_first and q < STAG:
                @pl.when(jsup > 0)
                def _():
                    finish_and_scatter(qq, dl_ref)
            else:
                finish_and_scatter(qq, dl_ref)
            if maybe_first and q < NBUF:
                @pl.when(jsup > 0)
                def _():
                    wait_scatter(b)
            else:
                wait_scatter(b)
            if q == STAG:
                @pl.when(jsup < nsup - 1)
                def _():
                    start_idx(jsup + 1, 1 - p)
            for jj in range(ECH // LANES):
                dlb[p][q][pl.ds(jj * LANES, LANES)] = \
                    ibd[p][pl.ds(q * ECH + jj * LANES, LANES)]
            pltpu.async_copy(h_hbm.at[srcb.at[pl.ds(q * ECH, ECH)]],
                             rows[b], sg[b])

    start_idx(0, 0)

    @pl.loop(0, nsup, step=2)
    def _(j):
        wait_idx(0)
        do_super(j, 0, True)
        wait_idx(1)
        do_super(j + 1, 1, False)

    for qs in (QPS - STAG, QPS - 1):
        finish_and_scatter(qs, dlb[1][qs])
    for b in range(NBUF):
        wait_scatter(b)

    plsc.subcore_barrier()

    @pl.when(s < NS - 1)
    def _():
        off = pl.multiple_of(s * STRIPE, 8)
        pltpu.sync_copy(acc.at[pl.ds(off, STRIPE)],
                        z_hbm.at[pl.ds(base_node + off, STRIPE)])

    @pl.when(s == NS - 1)
    def _():
        off = (NS - 1) * STRIPE
        pltpu.sync_copy(acc.at[pl.ds(off, LAST_STRIPE)],
                        z_hbm.at[pl.ds(base_node + off, LAST_STRIPE)])


@functools.partial(
    pl.kernel,
    out_type=jax.ShapeDtypeStruct((NC * 4 * M, D), jnp.float32),
    mesh=_mesh,
    compiler_params=_sc_params,
    scratch_types=[
        pltpu.VMEM_SHARED((MP, D), jnp.float32),
        pltpu.VMEM_SHARED((MP, D), jnp.float32),
        pltpu.VMEM_SHARED((MP, D), jnp.float32),
        pltpu.VMEM_SHARED((MP, D), jnp.float32),
        pltpu.VMEM((CH,), jnp.int32),
        pltpu.VMEM((CH,), jnp.int32),
        pltpu.VMEM((CH, D), jnp.float32),
        pltpu.VMEM((CH, D), jnp.float32),
    ],
)
def _pool_kernel(h1_hbm, h2_hbm, h3_hbm, mol_hbm, zeros_hbm, ones_hbm,
                 g_hbm, acc0, acc1, acc2, acc3, molv, idxv, rows, onesv):
    c = lax.axis_index("c")
    s = lax.axis_index("s")
    base_node = c * HALF
    accs = (acc0, acc1, acc2, acc3)

    zoff = pl.multiple_of(s * ZSTRIPE, 8)
    for a in accs:
        pltpu.sync_copy(zeros_hbm, a.at[pl.ds(zoff, ZSTRIPE)])
    pltpu.sync_copy(ones_hbm, onesv)
    plsc.subcore_barrier()

    @pl.loop(s, NCHUNK_P, step=NS)
    def _(ci):
        start = ci * CH
        b = pl.multiple_of(jnp.minimum(start, HALF - CH), 8)
        pltpu.sync_copy(mol_hbm.at[pl.ds(base_node + b, CH)], molv)

        @pl.loop(0, CH, step=LANES)
        def _(j):
            m = molv[pl.ds(j, LANES)]
            pos = b + j + lax.iota(jnp.int32, LANES)
            ok = pos >= start
            idxv[pl.ds(j, LANES)] = jnp.where(ok, m, MTRASH)

        for a, h in ((acc0, h1_hbm), (acc1, h2_hbm), (acc2, h3_hbm)):
            pltpu.sync_copy(h.at[pl.ds(base_node + b, CH)], rows)
            pltpu.sync_copy(rows, a.at[idxv], add=True)
        pltpu.sync_copy(onesv, acc3.at[idxv], add=True)

    plsc.subcore_barrier()

    woff = pl.multiple_of(s * WSTRIPE, 8)
    for l, a in enumerate(accs):
        out_off = (c * 4 + l) * M + woff
        pltpu.sync_copy(a.at[pl.ds(woff, WSTRIPE)],
                        g_hbm.at[pl.ds(out_off, WSTRIPE)])


_MLP_BLK = 400


def _mlp_body(z_ref, w1_ref, b1_ref, w2_ref, b2_ref, o_ref):
    z = z_ref[...]
    t = jnp.maximum(
        jnp.dot(z, w1_ref[...], preferred_element_type=jnp.float32)
        + b1_ref[...], 0.0)
    t = jnp.dot(t, w2_ref[...], preferred_element_type=jnp.float32) + b2_ref[...]
    o_ref[...] = jnp.maximum(t, 0.0)


def _mlp(z, w1, b1, w2, b2):
    return pl.pallas_call(
        _mlp_body,
        grid=(N // _MLP_BLK,),
        in_specs=[
            pl.BlockSpec((_MLP_BLK, D), lambda i: (i, 0)),
            pl.BlockSpec((D, D), lambda i: (0, 0)),
            pl.BlockSpec((1, D), lambda i: (0, 0)),
            pl.BlockSpec((D, D), lambda i: (0, 0)),
            pl.BlockSpec((1, D), lambda i: (0, 0)),
        ],
        out_specs=pl.BlockSpec((_MLP_BLK, D), lambda i: (i, 0)),
        out_shape=jax.ShapeDtypeStruct((N, D), jnp.float32),
    )(z, w1, b1.reshape(1, D), w2, b2.reshape(1, D))


def _tail_body(g_ref, jkw_ref, jkb_ref, ow_ref, ob_ref, o_ref):
    g0 = g_ref[0] + g_ref[4]
    g1 = g_ref[1] + g_ref[5]
    g2 = g_ref[2] + g_ref[6]
    counts = (g_ref[3] + g_ref[7])[:, 0:1]
    gj = (jnp.dot(g0, jkw_ref[0:D, :], preferred_element_type=jnp.float32)
          + jnp.dot(g1, jkw_ref[D:2 * D, :], preferred_element_type=jnp.float32)
          + jnp.dot(g2, jkw_ref[2 * D:3 * D, :], preferred_element_type=jnp.float32)
          + counts * jkb_ref[...])
    o_ref[...] = jnp.dot(gj, ow_ref[...], preferred_element_type=jnp.float32) \
        + ob_ref[...]


def _tail(g, jk_w, jk_b, out_w, out_b):
    owp = jnp.zeros((D, 128), jnp.float32).at[:, :out_w.shape[1]].set(out_w)
    obp = jnp.zeros((1, 128), jnp.float32).at[:, :out_b.shape[0]].set(out_b)
    full = pl.pallas_call(
        _tail_body,
        out_shape=jax.ShapeDtypeStruct((M, 128), jnp.float32),
    )(g.reshape(NC * 4, M, D), jk_w, jk_b.reshape(1, D), owp, obp)
    return full[:, :out_w.shape[1]]


def kernel(atom_ids, edge_indices, mol_ids, params):
    atom_ids = atom_ids.astype(jnp.int32)
    src = edge_indices[0].astype(jnp.int32)
    dst = edge_indices[1].astype(jnp.int32)
    mol_ids = mol_ids.astype(jnp.int32)

    srcp, dlp, cnt = _part_kernel(src, dst)
    x = _emb_kernel(params['emb'], atom_ids)
    hs = []
    h = x
    for layer in params['gin']:
        z = _msg_kernel(h, srcp, dlp, cnt)
        h = _mlp(z, layer['W1'], layer['b1'], layer['W2'], layer['b2'])
        hs.append(h)

    zeros = jnp.zeros((ZSTRIPE, D), jnp.float32)
    ones = jnp.ones((CH, D), jnp.float32)
    g = _pool_kernel(hs[0], hs[1], hs[2], mol_ids, zeros, ones)
    return _tail(g, params['jk_W'], params['jk_b'], params['out_W'],
                 params['out_b'])


# speedup vs baseline: 1.5301x; 1.5301x over previous
import functools

import jax
import jax.numpy as jnp
from jax import lax
from jax.experimental import pallas as pl
from jax.experimental.pallas import tpu as pltpu
from jax.experimental.pallas import tpu_sc as plsc

N = 50000
E = 800000
D = 64
M = 1024
NC = 2
NS = 16
LANES = 16

HALF = N // NC
STRIPE = 1560
SP_ROWS = 25008
LAST_STRIPE = HALF - (NS - 1) * STRIPE
TRASH = HALF
CH = 128
NCHUNK_E = E // CH
NCHUNK_N = (N + CH - 1) // CH

MP = 1152
MTRASH = M
ZSTRIPE = MP // NS
WSTRIPE = M // NS
NCHUNK_P = (HALF + CH - 1) // CH

_mesh = plsc.VectorSubcoreMesh(core_axis_name="c", subcore_axis_name="s")
_sc_params = pltpu.CompilerParams(use_tc_tiling_on_sc=False)


@functools.partial(
    pl.kernel,
    out_type=jax.ShapeDtypeStruct((N, D), jnp.float32),
    mesh=_mesh,
    compiler_params=_sc_params,
    scratch_types=[
        pltpu.VMEM((CH,), jnp.int32),
        pltpu.VMEM((CH, D), jnp.float32),
    ],
)
def _emb_kernel(emb_hbm, ids_hbm, x_hbm, idv, rows):
    c = lax.axis_index("c")
    s = lax.axis_index("s")
    w = s * NC + c

    @pl.loop(w, NCHUNK_N, step=NC * NS)
    def _(ci):
        b = pl.multiple_of(jnp.minimum(ci * CH, N - CH), 8)
        pltpu.sync_copy(ids_hbm.at[pl.ds(b, CH)], idv)
        pltpu.sync_copy(emb_hbm.at[idv], rows)
        pltpu.sync_copy(rows, x_hbm.at[pl.ds(b, CH)])


SUB_E = E // NS
ECH = 96
SUP_E = 8 * ECH
NSUP = 66
SUP_CLAMP = SUB_E - SUP_E
QPS = SUP_E // ECH
NBUF = 4


@functools.partial(
    pl.kernel,
    out_type=jax.ShapeDtypeStruct((N, D), jnp.float32),
    mesh=_mesh,
    compiler_params=_sc_params,
    scratch_types=[
        pltpu.VMEM_SHARED((SP_ROWS, D), jnp.float32),
        pltpu.VMEM((SUP_E,), jnp.int32),
        pltpu.VMEM((SUP_E,), jnp.int32),
        pltpu.VMEM((SUP_E,), jnp.int32),
        pltpu.VMEM((SUP_E,), jnp.int32),
        pltpu.VMEM((QPS, ECH), jnp.int32),
        pltpu.VMEM((ECH, D), jnp.float32),
        pltpu.VMEM((ECH, D), jnp.float32),
        pltpu.VMEM((ECH, D), jnp.float32),
        pltpu.VMEM((ECH, D), jnp.float32),
        pltpu.SemaphoreType.DMA,
        pltpu.SemaphoreType.DMA,
        pltpu.SemaphoreType.DMA,
        pltpu.SemaphoreType.DMA,
        pltpu.SemaphoreType.DMA,
        pltpu.SemaphoreType.DMA,
        pltpu.SemaphoreType.DMA,
        pltpu.SemaphoreType.DMA,
        pltpu.SemaphoreType.DMA,
        pltpu.SemaphoreType.DMA,
    ],
)
def _msg_kernel(h_hbm, src_hbm, dst_hbm, z_hbm, acc,
                ibs0, ibd0, ibs1, ibd1, dlv, r0, r1, r2, r3,
                si0, si1, sg0, sg1, sg2, sg3, sc0, sc1, sc2, sc3):
    c = lax.axis_index("c")
    s = lax.axis_index("s")
    base_node = c * HALF
    ebase = s * SUB_E
    rows = (r0, r1, r2, r3)
    sg = (sg0, sg1, sg2, sg3)
    scm = (sc0, sc1, sc2, sc3)
    ib = ((ibs0, ibd0, si0), (ibs1, ibd1, si1))
    lane = lax.iota(jnp.int32, LANES)

    @pl.when(s < NS - 1)
    def _():
        off = pl.multiple_of(s * STRIPE, 8)
        pltpu.sync_copy(h_hbm.at[pl.ds(base_node + off, STRIPE)],
                        acc.at[pl.ds(off, STRIPE)])

    @pl.when(s == NS - 1)
    def _():
        off = (NS - 1) * STRIPE
        pltpu.sync_copy(h_hbm.at[pl.ds(base_node + off, LAST_STRIPE)],
                        acc.at[pl.ds(off, LAST_STRIPE)])

    plsc.subcore_barrier()

    def sup_off(jsup):
        return pl.multiple_of(
            ebase + jnp.minimum(jsup * SUP_E, SUP_CLAMP), 8)

    def start_idx(jsup, p):
        off = sup_off(jsup)
        pltpu.async_copy(src_hbm.at[pl.ds(off, SUP_E)], ib[p][0], ib[p][2])
        pltpu.async_copy(dst_hbm.at[pl.ds(off, SUP_E)], ib[p][1], ib[p][2])

    def wait_idx(p):
        pltpu.make_async_copy(src_hbm.at[pl.ds(0, SUP_E)], ib[p][0],
                              ib[p][2]).wait()
        pltpu.make_async_copy(dst_hbm.at[pl.ds(0, SUP_E)], ib[p][1],
                              ib[p][2]).wait()

    def wait_scatter(b):
        pltpu.make_async_copy(rows[b], acc.at[dlv.at[0]], scm[b]).wait()

    def finish_and_scatter(qs):
        b = qs % NBUF
        pltpu.make_async_copy(h_hbm.at[ibs0.at[pl.ds(0, ECH)]],
                              rows[b], sg[b]).wait()
        pltpu.async_copy(rows[b], acc.at[dlv.at[qs]], scm[b], add=True)

    STAG = 2

    def do_super(jsup, p, maybe_first):
        off = sup_off(jsup)
        start_abs = ebase + jsup * SUP_E
        srcb, dstb, _ = ib[p]
        for q in range(QPS):
            b = q % NBUF
            qq = (q - STAG) % QPS
            if maybe_first and q < STAG:
                @pl.when(jsup > 0)
                def _():
                    finish_and_scatter(qq)
            else:
                finish_and_scatter(qq)
            if maybe_first and q < NBUF:
                @pl.when(jsup > 0)
                def _():
                    wait_scatter(b)
            else:
                wait_scatter(b)
            if q == STAG:
                @pl.when(jsup < NSUP - 1)
                def _():
                    start_idx(jsup + 1, 1 - p)
            for jj in range(ECH // LANES):
                d = dstb[pl.ds(q * ECH + jj * LANES, LANES)]
                dl = d - base_node
                t = start_abs - off - (q * ECH + jj * LANES)
                ok = (dl >= 0) & (dl < HALF) & (lane >= t)
                dlv[q, pl.ds(jj * LANES, LANES)] = jnp.where(ok, dl, TRASH)
            pltpu.async_copy(h_hbm.at[srcb.at[pl.ds(q * ECH, ECH)]],
                             rows[b], sg[b])

    start_idx(0, 0)

    @pl.loop(0, NSUP, step=2)
    def _(j):
        wait_idx(0)
        do_super(j, 0, True)
        wait_idx(1)
        do_super(j + 1, 1, False)

    for qs in (QPS - STAG, QPS - 1):
        finish_and_scatter(qs)
    for b in range(NBUF):
        wait_scatter(b)

    plsc.subcore_barrier()

    @pl.when(s < NS - 1)
    def _():
        off = pl.multiple_of(s * STRIPE, 8)
        pltpu.sync_copy(acc.at[pl.ds(off, STRIPE)],
                        z_hbm.at[pl.ds(base_node + off, STRIPE)])

    @pl.when(s == NS - 1)
    def _():
        off = (NS - 1) * STRIPE
        pltpu.sync_copy(acc.at[pl.ds(off, LAST_STRIPE)],
                        z_hbm.at[pl.ds(base_node + off, LAST_STRIPE)])


@functools.partial(
    pl.kernel,
    out_type=jax.ShapeDtypeStruct((NC * 4 * M, D), jnp.float32),
    mesh=_mesh,
    compiler_params=_sc_params,
    scratch_types=[
        pltpu.VMEM_SHARED((MP, D), jnp.float32),
        pltpu.VMEM_SHARED((MP, D), jnp.float32),
        pltpu.VMEM_SHARED((MP, D), jnp.float32),
        pltpu.VMEM_SHARED((MP, D), jnp.float32),
        pltpu.VMEM((CH,), jnp.int32),
        pltpu.VMEM((CH,), jnp.int32),
        pltpu.VMEM((CH, D), jnp.float32),
        pltpu.VMEM((CH, D), jnp.float32),
    ],
)
def _pool_kernel(h1_hbm, h2_hbm, h3_hbm, mol_hbm, zeros_hbm, ones_hbm,
                 g_hbm, acc0, acc1, acc2, acc3, molv, idxv, rows, onesv):
    c = lax.axis_index("c")
    s = lax.axis_index("s")
    base_node = c * HALF
    accs = (acc0, acc1, acc2, acc3)

    zoff = pl.multiple_of(s * ZSTRIPE, 8)
    for a in accs:
        pltpu.sync_copy(zeros_hbm, a.at[pl.ds(zoff, ZSTRIPE)])
    pltpu.sync_copy(ones_hbm, onesv)
    plsc.subcore_barrier()

    @pl.loop(s, NCHUNK_P, step=NS)
    def _(ci):
        start = ci * CH
        b = pl.multiple_of(jnp.minimum(start, HALF - CH), 8)
        pltpu.sync_copy(mol_hbm.at[pl.ds(base_node + b, CH)], molv)

        @pl.loop(0, CH, step=LANES)
        def _(j):
            m = molv[pl.ds(j, LANES)]
            pos = b + j + lax.iota(jnp.int32, LANES)
            ok = pos >= start
            idxv[pl.ds(j, LANES)] = jnp.where(ok, m, MTRASH)

        for a, h in ((acc0, h1_hbm), (acc1, h2_hbm), (acc2, h3_hbm)):
            pltpu.sync_copy(h.at[pl.ds(base_node + b, CH)], rows)
            pltpu.sync_copy(rows, a.at[idxv], add=True)
        pltpu.sync_copy(onesv, acc3.at[idxv], add=True)

    plsc.subcore_barrier()

    woff = pl.multiple_of(s * WSTRIPE, 8)
    for l, a in enumerate(accs):
        out_off = (c * 4 + l) * M + woff
        pltpu.sync_copy(a.at[pl.ds(woff, WSTRIPE)],
                        g_hbm.at[pl.ds(out_off, WSTRIPE)])


_MLP_BLK = 400


def _mlp_body(z_ref, w1_ref, b1_ref, w2_ref, b2_ref, o_ref):
    z = z_ref[...]
    t = jnp.maximum(
        jnp.dot(z, w1_ref[...], preferred_element_type=jnp.float32)
        + b1_ref[...], 0.0)
    t = jnp.dot(t, w2_ref[...], preferred_element_type=jnp.float32) + b2_ref[...]
    o_ref[...] = jnp.maximum(t, 0.0)


def _mlp(z, w1, b1, w2, b2):
    return pl.pallas_call(
        _mlp_body,
        grid=(N // _MLP_BLK,),
        in_specs=[
            pl.BlockSpec((_MLP_BLK, D), lambda i: (i, 0)),
            pl.BlockSpec((D, D), lambda i: (0, 0)),
            pl.BlockSpec((1, D), lambda i: (0, 0)),
            pl.BlockSpec((D, D), lambda i: (0, 0)),
            pl.BlockSpec((1, D), lambda i: (0, 0)),
        ],
        out_specs=pl.BlockSpec((_MLP_BLK, D), lambda i: (i, 0)),
        out_shape=jax.ShapeDtypeStruct((N, D), jnp.float32),
    )(z, w1, b1.reshape(1, D), w2, b2.reshape(1, D))


def _tail_body(g_ref, jkw_ref, jkb_ref, ow_ref, ob_ref, o_ref):
    g0 = g_ref[0] + g_ref[4]
    g1 = g_ref[1] + g_ref[5]
    g2 = g_ref[2] + g_ref[6]
    counts = (g_ref[3] + g_ref[7])[:, 0:1]
    gj = (jnp.dot(g0, jkw_ref[0:D, :], preferred_element_type=jnp.float32)
          + jnp.dot(g1, jkw_ref[D:2 * D, :], preferred_element_type=jnp.float32)
          + jnp.dot(g2, jkw_ref[2 * D:3 * D, :], preferred_element_type=jnp.float32)
          + counts * jkb_ref[...])
    o_ref[...] = jnp.dot(gj, ow_ref[...], preferred_element_type=jnp.float32) \
        + ob_ref[...]


def _tail(g, jk_w, jk_b, out_w, out_b):
    owp = jnp.zeros((D, 128), jnp.float32).at[:, :out_w.shape[1]].set(out_w)
    obp = jnp.zeros((1, 128), jnp.float32).at[:, :out_b.shape[0]].set(out_b)
    full = pl.pallas_call(
        _tail_body,
        out_shape=jax.ShapeDtypeStruct((M, 128), jnp.float32),
    )(g.reshape(NC * 4, M, D), jk_w, jk_b.reshape(1, D), owp, obp)
    return full[:, :out_w.shape[1]]


def kernel(atom_ids, edge_indices, mol_ids, params):
    atom_ids = atom_ids.astype(jnp.int32)
    src = edge_indices[0].astype(jnp.int32)
    dst = edge_indices[1].astype(jnp.int32)
    mol_ids = mol_ids.astype(jnp.int32)

    x = _emb_kernel(params['emb'], atom_ids)
    hs = []
    h = x
    for layer in params['gin']:
        z = _msg_kernel(h, src, dst)
        h = _mlp(z, layer['W1'], layer['b1'], layer['W2'], layer['b2'])
        hs.append(h)

    zeros = jnp.zeros((ZSTRIPE, D), jnp.float32)
    ones = jnp.ones((CH, D), jnp.float32)
    g = _pool_kernel(hs[0], hs[1], hs[2], mol_ids, zeros, ones)
    return _tail(g, params['jk_W'], params['jk_b'], params['out_W'],
                 params['out_b'])
